# Initial kernel scaffold; baseline (speedup 1.0000x reference)
import jax
import jax.numpy as jnp
from jax.experimental import pallas as pl


def kernel(pred, target):
    raise NotImplementedError("write your pallas kernel here")


# speedup vs baseline: 18.0478x; 18.0478x over previous
import functools

import jax
import jax.numpy as jnp
from jax.experimental import pallas as pl

_K = 8


def _block_kernel(xp_ref, xpT_ref, xt_ref, xtT_ref,
                  dens_ref, sim_ref, cov_ref, *, n_pts, blk_rows):
    b = pl.program_id(0)
    i = pl.program_id(1)

    @pl.when(jnp.logical_and(b == 0, i == 0))
    def _init():
        zero = jnp.zeros((1, 1), jnp.float32)
        dens_ref[:, :] = zero
        sim_ref[:, :] = zero
        cov_ref[:, :] = zero

    row0 = i * blk_rows
    col_ids = jax.lax.broadcasted_iota(jnp.int32, (blk_rows, n_pts), 1)

    def select_neighbors(x_ref, xT_ref):
        xa = x_ref[0, :, :]
        xaT = xT_ref[0, :, :]
        xb = x_ref[0, pl.ds(row0, blk_rows), :]
        sqb = jnp.sum(xb * xb, axis=1, keepdims=True)
        sqa = jnp.sum(xaT * xaT, axis=0, keepdims=True)
        cross = jnp.dot(xb, xaT, preferred_element_type=jnp.float32)
        d2 = sqb + sqa - 2.0 * cross
        dist = jnp.sqrt(jnp.maximum(d2, 1e-12))
        dists, nbrs = [], []
        for k in range(_K + 1):
            m = jnp.min(dist, axis=1, keepdims=True)
            cand = jnp.where(dist == m, col_ids, n_pts)
            idx = jnp.min(cand, axis=1, keepdims=True)
            onehot = col_ids == idx
            if k > 0:
                dists.append(m)
                nbrs.append(jnp.dot(onehot.astype(jnp.float32), xa,
                                    preferred_element_type=jnp.float32))
            dist = jnp.where(onehot, jnp.float32(jnp.inf), dist)
        return xb, dists, nbrs

    xpb, pdists, pnbrs = select_neighbors(xp_ref, xpT_ref)
    xtb, tdists, tnbrs = select_neighbors(xt_ref, xtT_ref)

    comps = [(0, 0), (1, 1), (2, 2), (0, 1), (0, 2), (1, 2)]
    pcov = [jnp.zeros((blk_rows, 1), jnp.float32) for _ in comps]
    tcov = [jnp.zeros((blk_rows, 1), jnp.float32) for _ in comps]
    sim_acc = jnp.zeros((blk_rows, 1), jnp.float32)
    for k in range(_K):
        pv = pnbrs[k] - xpb
        tv = tnbrs[k] - xtb
        pn = jnp.maximum(jnp.sqrt(jnp.sum(pv * pv, axis=1, keepdims=True)),
                         1e-12)
        tn = jnp.maximum(jnp.sqrt(jnp.sum(tv * tv, axis=1, keepdims=True)),
                         1e-12)
        sim_acc += jnp.sum((pv / pn) * (tv / tn), axis=1, keepdims=True)
        for c, (a, bb) in enumerate(comps):
            pcov[c] = pcov[c] + pv[:, a:a + 1] * pv[:, bb:bb + 1]
            tcov[c] = tcov[c] + tv[:, a:a + 1] * tv[:, bb:bb + 1]

    inv_k = jnp.float32(1.0 / _K)
    pdens = sum(pdists) * inv_k
    tdens = sum(tdists) * inv_k
    ddiff = pdens - tdens

    dc = [pcov[c] - tcov[c] for c in range(6)]
    fro2 = (dc[0] * dc[0] + dc[1] * dc[1] + dc[2] * dc[2]
            + 2.0 * (dc[3] * dc[3] + dc[4] * dc[4] + dc[5] * dc[5]))
    cov_row = jnp.sqrt(fro2) * inv_k

    dens_ref[:, :] += jnp.sum(ddiff * ddiff, axis=0, keepdims=True)
    sim_ref[:, :] += jnp.sum(sim_acc, axis=0, keepdims=True)
    cov_ref[:, :] += jnp.sum(cov_row, axis=0, keepdims=True)


@jax.jit
def kernel(pred, target):
    B, N, _ = pred.shape
    R = 256
    nb = N // R
    predT = jnp.swapaxes(pred, 1, 2)
    targetT = jnp.swapaxes(target, 1, 2)

    full = pl.BlockSpec((1, N, 3), lambda b, i: (b, 0, 0))
    fullT = pl.BlockSpec((1, 3, N), lambda b, i: (b, 0, 0))
    scalar = pl.BlockSpec((1, 1), lambda b, i: (0, 0))

    dens, sim, cov = pl.pallas_call(
        functools.partial(_block_kernel, n_pts=N, blk_rows=R),
        grid=(B, nb),
        in_specs=[full, fullT, full, fullT],
        out_specs=[scalar, scalar, scalar],
        out_shape=[jax.ShapeDtypeStruct((1, 1), jnp.float32)] * 3,
    )(pred, predT, target, targetT)

    dens_t = dens[0, 0]
    sim_t = sim[0, 0]
    cov_t = cov[0, 0]
    alpha = jnp.float32(0.5)
    loss = (dens_t / N
            + alpha * (B - sim_t / (N * _K))
            + (1.0 - alpha) * cov_t / N) / B
    return loss
